# Initial kernel scaffold; baseline (speedup 1.0000x reference)
import jax
import jax.numpy as jnp
from jax.experimental import pallas as pl


def kernel(x, adj, w, W1, W2):
    raise NotImplementedError("write your pallas kernel here")


# speedup vs baseline: 4.2756x; 4.2756x over previous
import functools

import jax
import jax.numpy as jnp
from jax import lax
from jax.experimental import pallas as pl
from jax.experimental.pallas import tpu as pltpu
from jax.experimental.pallas import tpu_sc as plsc

N = 10000
E = 320000
D = 128
NC, NS, L = 2, 16, 16
NW = NC * NS
EPT = E // NW
C = 80
NCHUNK = EPT // C
RSTAGE = 80
NB = N // RSTAGE
BPT = -(-NB // NS)

_mesh = plsc.VectorSubcoreMesh(core_axis_name="c", subcore_axis_name="s")


@functools.partial(
    pl.kernel,
    out_type=jax.ShapeDtypeStruct((NC, N, D), jnp.float32),
    mesh=_mesh,
    scratch_types=[
        pltpu.VMEM((C,), jnp.int32),
        pltpu.VMEM((C,), jnp.int32),
        pltpu.VMEM((C,), jnp.float32),
        pltpu.VMEM((C, D), jnp.float32),
        pltpu.VMEM((RSTAGE, D), jnp.float32),
        pltpu.VMEM_SHARED((N, D), jnp.float32),
        pltpu.SemaphoreType.DMA,
    ],
)
def _sc_scatter(x_hbm, src_hbm, dst_hbm, w_hbm, out_hbm,
                sidx, didx, wv, rows, stage, acc, sem):
    cid = lax.axis_index("c")
    sid = lax.axis_index("s")
    wid = sid * NC + cid

    def _zrow(i, _):
        for f in range(D // L):
            stage[i, pl.ds(f * L, L)] = jnp.zeros((L,), jnp.float32)
        return 0
    lax.fori_loop(0, RSTAGE, _zrow, 0)
    for j in range(BPT):
        b = j * NS + sid
        @pl.when(b < NB)
        def _():
            pltpu.sync_copy(stage, acc.at[pl.ds(b * RSTAGE, RSTAGE)])
    plsc.subcore_barrier()

    base = wid * EPT

    def _chunk(ci, _):
        off = base + ci * C
        pltpu.sync_copy(src_hbm.at[pl.ds(off, C)], sidx)
        pltpu.sync_copy(dst_hbm.at[pl.ds(off, C)], didx)
        pltpu.sync_copy(w_hbm.at[pl.ds(off, C)], wv)
        pltpu.async_copy(x_hbm.at[sidx], rows, sem).wait()

        def _group(g, _):
            w16 = wv[pl.ds(g * L, L)]
            for j in range(L):
                we = w16[j]
                e = g * L + j
                for f in range(D // L):
                    rows[e, pl.ds(f * L, L)] = rows[e, pl.ds(f * L, L)] * we
            return 0
        lax.fori_loop(0, C // L, _group, 0)

        pltpu.sync_copy(rows, acc.at[didx], add=True)
        return 0
    lax.fori_loop(0, NCHUNK, _chunk, 0)

    plsc.subcore_barrier()
    for j in range(BPT):
        b = j * NS + sid
        @pl.when(b < NB)
        def _():
            r0 = b * RSTAGE
            pltpu.sync_copy(acc.at[pl.ds(r0, RSTAGE)], stage)
            pltpu.sync_copy(stage, out_hbm.at[cid, pl.ds(r0, RSTAGE)])


def _tc_fuse_kernel(p_ref, w_ref, o_ref):
    s = p_ref[0] + p_ref[1]
    o_ref[...] = jnp.maximum(
        jnp.dot(s, w_ref[...], preferred_element_type=jnp.float32), 0.0)


_tc_fuse = pl.pallas_call(
    _tc_fuse_kernel,
    out_shape=jax.ShapeDtypeStruct((N, D), jnp.float32),
)


def kernel(x, adj, w, W1, W2):
    adj = adj.astype(jnp.int32)
    src, dst = adj[0], adj[1]
    p1 = _sc_scatter(x, src, dst, w)
    x1 = _tc_fuse(p1, W1)
    p2 = _sc_scatter(x1, src, dst, w)
    return _tc_fuse(p2, W2)


# speedup vs baseline: 11.3859x; 2.6630x over previous
import functools

import jax
import jax.numpy as jnp
from jax import lax
from jax.experimental import pallas as pl
from jax.experimental.pallas import tpu as pltpu
from jax.experimental.pallas import tpu_sc as plsc

N = 10000
E = 320000
D = 128
NC, NS, L = 2, 16, 16
NW = NC * NS
C = 80
NCHUNK = 125
EPT = NCHUNK * C
RSTAGE = 80
NB = N // RSTAGE
BPT = -(-NB // NS)

_mesh = plsc.VectorSubcoreMesh(core_axis_name="c", subcore_axis_name="s")


@functools.partial(
    pl.kernel,
    out_type=jax.ShapeDtypeStruct((NC, N, D), jnp.float32),
    mesh=_mesh,
    scratch_types=[
        pltpu.VMEM((EPT,), jnp.int32),
        pltpu.VMEM((EPT,), jnp.float32),
        pltpu.VMEM((2 * C,), jnp.int32),
        pltpu.VMEM((2, C), jnp.int32),
        pltpu.VMEM((2 * C, D), jnp.float32),
        pltpu.VMEM_SHARED((N, D), jnp.float32),
        pltpu.SemaphoreType.DMA,
        pltpu.SemaphoreType.DMA((2,)),
        pltpu.SemaphoreType.DMA((2,)),
    ],
)
def _sc_scatter(x_hbm, packed_hbm, w_hbm, out_hbm,
                pk, wv, sidx, didx, rows, acc, isem, gsem, ssem):
    cid = lax.axis_index("c")
    sid = lax.axis_index("s")
    wid = sid * NC + cid

    a1 = pltpu.async_copy(packed_hbm.at[wid], pk, isem)
    a2 = pltpu.async_copy(w_hbm.at[wid], wv, isem)

    def _zrow(i, _):
        for f in range(D // L):
            rows[i, pl.ds(f * L, L)] = jnp.zeros((L,), jnp.float32)
        return 0
    lax.fori_loop(0, RSTAGE, _zrow, 0)
    for j in range(BPT):
        b = j * NS + sid
        @pl.when(b < NB)
        def _():
            pltpu.sync_copy(rows.at[pl.ds(0, RSTAGE)],
                            acc.at[pl.ds(b * RSTAGE, RSTAGE)])
    a1.wait()
    a2.wait()

    def _unpack(ci, slot):
        for g in range(C // L):
            v = pk[pl.ds(ci * C + g * L, L)]
            sidx[pl.ds(slot * C + g * L, L)] = jnp.bitwise_and(v, 0xFFFF)
            didx[slot, pl.ds(g * L, L)] = lax.shift_right_logical(v, 16)

    _unpack(0, 0)
    pltpu.async_copy(x_hbm.at[sidx.at[pl.ds(0, C)]],
                     rows.at[pl.ds(0, C)], gsem.at[0])
    plsc.subcore_barrier()

    def _chunk(ci, _):
        p = lax.rem(ci, 2)
        q = 1 - p
        pof = p * C
        qof = q * C

        @pl.when(ci >= 1)
        def _():
            pltpu.make_async_copy(
                x_hbm.at[pl.ds(0, C)], rows.at[pl.ds(qof, C)],
                ssem.at[q]).wait()

        @pl.when(ci + 1 < NCHUNK)
        def _():
            _unpack(ci + 1, q)
            pltpu.async_copy(x_hbm.at[sidx.at[pl.ds(qof, C)]],
                             rows.at[pl.ds(qof, C)], gsem.at[q])

        pltpu.make_async_copy(
            x_hbm.at[pl.ds(0, C)], rows.at[pl.ds(pof, C)], gsem.at[p]).wait()

        def _group(g, _):
            w16 = wv[pl.ds(ci * C + g * L, L)]
            for j in range(L):
                we = w16[j]
                e = pof + g * L + j
                for f in range(D // L):
                    rows[e, pl.ds(f * L, L)] = rows[e, pl.ds(f * L, L)] * we
            return 0
        lax.fori_loop(0, C // L, _group, 0)

        pltpu.async_copy(rows.at[pl.ds(pof, C)], acc.at[didx.at[p]],
                         ssem.at[p], add=True)
        return 0
    lax.fori_loop(0, NCHUNK, _chunk, 0)

    pltpu.make_async_copy(
        x_hbm.at[pl.ds(0, C)], rows.at[pl.ds(0, C)], ssem.at[0]).wait()
    plsc.subcore_barrier()

    for j in range(BPT):
        b = j * NS + sid
        @pl.when(b < NB)
        def _():
            r0 = b * RSTAGE
            pltpu.sync_copy(acc.at[pl.ds(r0, RSTAGE)],
                            rows.at[pl.ds(0, RSTAGE)])
            pltpu.sync_copy(rows.at[pl.ds(0, RSTAGE)],
                            out_hbm.at[cid, pl.ds(r0, RSTAGE)])


def _tc_fuse_kernel(p_ref, w_ref, o_ref):
    s = p_ref[0] + p_ref[1]
    o_ref[...] = jnp.maximum(
        jnp.dot(s, w_ref[...], preferred_element_type=jnp.float32), 0.0)


_tc_fuse = pl.pallas_call(
    _tc_fuse_kernel,
    out_shape=jax.ShapeDtypeStruct((N, D), jnp.float32),
)


def kernel(x, adj, w, W1, W2):
    adj = adj.astype(jnp.int32)
    packed = (adj[0] | (adj[1] << 16)).reshape(NW, EPT)
    wp = w.reshape(NW, EPT)
    p1 = _sc_scatter(x, packed, wp)
    x1 = _tc_fuse(p1, W1)
    p2 = _sc_scatter(x1, packed, wp)
    return _tc_fuse(p2, W2)


# speedup vs baseline: 13.5756x; 1.1923x over previous
import functools

import jax
import jax.numpy as jnp
from jax import lax
from jax.experimental import pallas as pl
from jax.experimental.pallas import tpu as pltpu
from jax.experimental.pallas import tpu_sc as plsc

N = 10000
E = 320000
D = 128
NC, NS, L = 2, 16, 16
NW = NC * NS
C = 80
NCHUNK = 125
EPT = NCHUNK * C
NBUF = 4
RSTAGE = 80
NB = N // RSTAGE
BPT = -(-NB // NS)

_mesh = plsc.VectorSubcoreMesh(core_axis_name="c", subcore_axis_name="s")


@functools.partial(
    pl.kernel,
    out_type=jax.ShapeDtypeStruct((NC, N, D), jnp.float32),
    mesh=_mesh,
    scratch_types=[
        pltpu.VMEM((NBUF, C), jnp.int32),
        pltpu.VMEM((NBUF, C), jnp.float32),
        pltpu.VMEM((NBUF * C,), jnp.int32),
        pltpu.VMEM((NBUF, C), jnp.int32),
        pltpu.VMEM((NBUF * C, D), jnp.float32),
        pltpu.VMEM_SHARED((N, D), jnp.float32),
        pltpu.SemaphoreType.DMA((NBUF,)),
        pltpu.SemaphoreType.DMA((NBUF,)),
        pltpu.SemaphoreType.DMA((NBUF,)),
    ],
)
def _sc_scatter(x_hbm, pk_hbm, w_hbm, out_hbm,
                ebuf, wbuf, sidx, didx, rows, acc, esem, gsem, ssem):
    cid = lax.axis_index("c")
    sid = lax.axis_index("s")
    wid = sid * NC + cid
    ebase = wid * NCHUNK

    def _fetch(k, slot):
        pltpu.async_copy(pk_hbm.at[pl.ds((ebase + k) * C, C)],
                         ebuf.at[slot], esem.at[slot])
        pltpu.async_copy(w_hbm.at[pl.ds((ebase + k) * C, C)],
                         wbuf.at[slot], esem.at[slot])

    def _wait_fetch(slot):
        pltpu.make_async_copy(pk_hbm.at[pl.ds(0, C)], ebuf.at[slot],
                              esem.at[slot]).wait()
        pltpu.make_async_copy(w_hbm.at[pl.ds(0, C)], wbuf.at[slot],
                              esem.at[slot]).wait()

    def _unpack(k, slot):
        for g in range(C // L):
            v = ebuf[slot, pl.ds(g * L, L)]
            sidx[pl.ds(slot * C + g * L, L)] = jnp.bitwise_and(v, 0xFFFF)
            didx[slot, pl.ds(g * L, L)] = lax.shift_right_logical(v, 16)

    def _gather(k, slot):
        pltpu.async_copy(x_hbm.at[sidx.at[pl.ds(slot * C, C)]],
                         rows.at[pl.ds(slot * C, C)], gsem.at[slot])

    def _wait_rows_sem(sem, slot):
        pltpu.make_async_copy(x_hbm.at[pl.ds(0, C)],
                              rows.at[pl.ds(slot * C, C)], sem.at[slot]).wait()

    for k in range(3):
        _fetch(k, k)

    def _zrow(i, _):
        for f in range(D // L):
            rows[i, pl.ds(f * L, L)] = jnp.zeros((L,), jnp.float32)
        return 0
    lax.fori_loop(0, RSTAGE, _zrow, 0)
    for j in range(BPT):
        b = j * NS + sid
        @pl.when(b < NB)
        def _():
            pltpu.sync_copy(rows.at[pl.ds(0, RSTAGE)],
                            acc.at[pl.ds(b * RSTAGE, RSTAGE)])

    for k in range(2):
        _wait_fetch(k)
        _unpack(k, k)
        _gather(k, k)
    plsc.subcore_barrier()

    def _chunk(ci, _):
        p = lax.rem(ci, NBUF)

        @pl.when(ci + 3 < NCHUNK)
        def _():
            _fetch(ci + 3, lax.rem(ci + 3, NBUF))

        @pl.when(ci + 2 < NCHUNK)
        def _():
            s2 = lax.rem(ci + 2, NBUF)

            @pl.when(ci >= 2)
            def _():
                _wait_rows_sem(ssem, s2)
            _wait_fetch(s2)
            _unpack(ci + 2, s2)
            _gather(ci + 2, s2)

        _wait_rows_sem(gsem, p)

        def _group(g, _):
            w16 = wbuf[p, pl.ds(g * L, L)]
            for j in range(L):
                we = w16[j]
                e = p * C + g * L + j
                for f in range(D // L):
                    rows[e, pl.ds(f * L, L)] = rows[e, pl.ds(f * L, L)] * we
            return 0
        lax.fori_loop(0, C // L, _group, 0)

        pltpu.async_copy(rows.at[pl.ds(p * C, C)], acc.at[didx.at[p]],
                         ssem.at[p], add=True)
        return 0
    lax.fori_loop(0, NCHUNK, _chunk, 0)

    for k in range(NCHUNK - NBUF, NCHUNK):
        _wait_rows_sem(ssem, k % NBUF)
    plsc.subcore_barrier()

    for j in range(BPT):
        b = j * NS + sid
        @pl.when(b < NB)
        def _():
            r0 = b * RSTAGE
            pltpu.sync_copy(acc.at[pl.ds(r0, RSTAGE)],
                            rows.at[pl.ds(0, RSTAGE)])
            pltpu.sync_copy(rows.at[pl.ds(0, RSTAGE)],
                            out_hbm.at[cid, pl.ds(r0, RSTAGE)])


def _tc_fuse_kernel(p_ref, w_ref, o_ref):
    s = p_ref[0] + p_ref[1]
    o_ref[...] = jnp.maximum(
        jnp.dot(s, w_ref[...], preferred_element_type=jnp.float32), 0.0)


_tc_fuse = pl.pallas_call(
    _tc_fuse_kernel,
    out_shape=jax.ShapeDtypeStruct((N, D), jnp.float32),
)


def kernel(x, adj, w, W1, W2):
    adj = adj.astype(jnp.int32)
    packed = adj[0] | (adj[1] << 16)
    p1 = _sc_scatter(x, packed, w)
    x1 = _tc_fuse(p1, W1)
    p2 = _sc_scatter(x1, packed, w)
    return _tc_fuse(p2, W2)
